# baseline (speedup 1.0000x reference)
import functools

import jax
import jax.numpy as jnp
from jax import lax
from jax.experimental import pallas as pl
from jax.experimental.pallas import tpu as pltpu
from jax.experimental.pallas import tpu_sc as plsc

_INPUT_DIM = 100000
_LATENT_DIM = 64
_BATCH = 1024

_LANES = 16


def _sc_gather_sigmoid(x, encoder_weight):
    info = plsc.get_sparse_core_info()
    nc, ns = info.num_cores, info.num_subcores
    nw = nc * ns
    b_per_w = _BATCH // nw
    mesh = plsc.VectorSubcoreMesh(core_axis_name="c", subcore_axis_name="s")

    @functools.partial(
        pl.kernel,
        mesh=mesh,
        compiler_params=pltpu.CompilerParams(use_tc_tiling_on_sc=False),
        out_type=jax.ShapeDtypeStruct((_BATCH, _LATENT_DIM), jnp.float32),
        scratch_types=[
            pltpu.VMEM((b_per_w,), jnp.int32),
            pltpu.VMEM((b_per_w, _LATENT_DIM), jnp.float32),
            pltpu.SemaphoreType.DMA,
        ],
    )
    def body(x_hbm, table_hbm, out_hbm, idx_v, rows_v, sem):
        wid = lax.axis_index("s") * nc + lax.axis_index("c")
        base = wid * b_per_w
        pltpu.sync_copy(x_hbm.at[pl.ds(base, b_per_w)], idx_v)
        pltpu.async_copy(table_hbm.at[idx_v], rows_v, sem).wait()
        for i in range(b_per_w):
            for j in range(_LATENT_DIM // _LANES):
                v = rows_v[i, pl.ds(j * _LANES, _LANES)]
                rows_v[i, pl.ds(j * _LANES, _LANES)] = 1.0 / (1.0 + jnp.exp(-v))
        pltpu.sync_copy(rows_v, out_hbm.at[pl.ds(base, b_per_w)])

    return body(x, encoder_weight)


_TILE_N = 2048
_MAIN_STEPS = _INPUT_DIM // _TILE_N
_TAIL_BLOCK = _MAIN_STEPS
_NBUF = 4


def _mm_body(h_ref, d_ref, o_ref, buf, sems):
    j = pl.program_id(0)
    slot = jax.lax.rem(j, _NBUF)

    @pl.when(j >= _NBUF)
    def _wait_prev():
        pltpu.make_async_copy(
            buf.at[slot],
            o_ref.at[:, pl.ds(0, _TILE_N)],
            sems.at[slot],
        ).wait()

    acc = jnp.dot(h_ref[...], d_ref[...], preferred_element_type=jnp.float32)
    buf[slot] = 1.0 / (1.0 + jnp.exp(-acc))

    pltpu.make_async_copy(
        buf.at[slot],
        o_ref.at[:, pl.ds(j * _TILE_N, _TILE_N)],
        sems.at[slot],
    ).start()

    @pl.when(j == _MAIN_STEPS - 1)
    def _drain_all():
        for k in range(_MAIN_STEPS - _NBUF, _MAIN_STEPS):
            pltpu.make_async_copy(
                buf.at[k % _NBUF],
                o_ref.at[:, pl.ds(0, _TILE_N)],
                sems.at[k % _NBUF],
            ).wait()


def _tail_body(h_ref, d_ref, prev_ref, o_ref):
    del prev_ref
    acc = jnp.dot(h_ref[...], d_ref[...], preferred_element_type=jnp.float32)
    o_ref[...] = 1.0 / (1.0 + jnp.exp(-acc))


def _tc_decode(h, decoder_weight):
    main = pl.pallas_call(
        _mm_body,
        grid=(_MAIN_STEPS,),
        in_specs=[
            pl.BlockSpec((_BATCH, _LATENT_DIM), lambda j: (0, 0)),
            pl.BlockSpec((_LATENT_DIM, _TILE_N), lambda j: (0, j)),
        ],
        out_specs=pl.BlockSpec(memory_space=pl.ANY),
        out_shape=jax.ShapeDtypeStruct((_BATCH, _INPUT_DIM), jnp.float32),
        scratch_shapes=[
            pltpu.VMEM((_NBUF, _BATCH, _TILE_N), jnp.float32),
            pltpu.SemaphoreType.DMA((_NBUF,)),
        ],
    )(h, decoder_weight)
    return pl.pallas_call(
        _tail_body,
        grid=(1,),
        in_specs=[
            pl.BlockSpec((_BATCH, _LATENT_DIM), lambda j: (0, 0)),
            pl.BlockSpec((_LATENT_DIM, _TILE_N), lambda j: (0, _TAIL_BLOCK)),
            pl.BlockSpec(memory_space=pl.ANY),
        ],
        out_specs=pl.BlockSpec((_BATCH, _TILE_N), lambda j: (0, _TAIL_BLOCK)),
        out_shape=jax.ShapeDtypeStruct((_BATCH, _INPUT_DIM), jnp.float32),
        input_output_aliases={2: 0},
    )(h, decoder_weight, main)


def kernel(x, encoder_weight, decoder_weight):
    h = _sc_gather_sigmoid(x.astype(jnp.int32), encoder_weight)
    return _tc_decode(h, decoder_weight)
